# Initial kernel scaffold; baseline (speedup 1.0000x reference)
import functools

import jax
import jax.numpy as jnp
from jax import lax
from jax.experimental import pallas as pl
from jax.experimental.pallas import tpu as pltpu
from jax.experimental.pallas import tpu_sc as plsc

N = 10000
E = 320000
NC = 2
NS = 16
NW = NC * NS
EPW = E // NW
CHUNK = 125
NCHUNK = EPW // CHUNK
RPT = N // NS
ZROWS = 125

_MESH = plsc.VectorSubcoreMesh(core_axis_name="c", subcore_axis_name="s")


def _selu(x):
    alpha = 1.6732632423543772848170429916717
    scale = 1.0507009873554804934193349852946
    return scale * jnp.where(x > 0, x, alpha * (jnp.exp(x) - 1.0))


def _deg_body(dst_hbm, out_hbm, dst_v, ones_v, acc):
    c = lax.axis_index("c")
    s = lax.axis_index("s")
    wid = c * NS + s
    r0 = s * RPT

    def zfill(i, carry):
        ones_v[i, :] = jnp.zeros((16,), jnp.float32)
        return carry

    lax.fori_loop(0, ZROWS, zfill, 0)
    for k in range(RPT // ZROWS):
        pltpu.sync_copy(ones_v, acc.at[pl.ds(r0 + k * ZROWS, ZROWS)])

    def fill(i, carry):
        ones_v[i, :] = jnp.ones((16,), jnp.float32)
        return carry

    lax.fori_loop(0, CHUNK, fill, 0)
    pltpu.sync_copy(dst_hbm.at[wid], dst_v)
    plsc.subcore_barrier()

    def body(j, carry):
        pltpu.sync_copy(ones_v, acc.at[dst_v.at[j]], add=True)
        return carry

    lax.fori_loop(0, NCHUNK, body, 0)
    plsc.subcore_barrier()
    pltpu.sync_copy(acc.at[pl.ds(r0, RPT)], out_hbm.at[c, pl.ds(r0, RPT)])


_deg_call = pl.kernel(
    _deg_body,
    out_type=jax.ShapeDtypeStruct((NC, N, 16), jnp.float32),
    mesh=_MESH,
    scratch_types=[
        pltpu.VMEM((NCHUNK, CHUNK), jnp.int32),
        pltpu.VMEM((CHUNK, 16), jnp.float32),
        pltpu.VMEM_SHARED((N, 16), jnp.float32),
    ],
)


def _agg_body(D, g_hbm, src_hbm, dst_hbm, out_hbm, src_v, dst_v, rows_v, zbuf,
              acc, sem):
    c = lax.axis_index("c")
    s = lax.axis_index("s")
    wid = c * NS + s
    r0 = s * RPT

    def zfill(i, carry):
        for t in range(D // 16):
            zbuf[i, pl.ds(t * 16, 16)] = jnp.zeros((16,), jnp.float32)
        return carry

    lax.fori_loop(0, ZROWS, zfill, 0)
    for k in range(RPT // ZROWS):
        pltpu.sync_copy(zbuf, acc.at[pl.ds(r0 + k * ZROWS, ZROWS)])
    pltpu.sync_copy(src_hbm.at[wid], src_v)
    pltpu.sync_copy(dst_hbm.at[wid], dst_v)
    plsc.subcore_barrier()

    def body(j, carry):
        pltpu.async_copy(g_hbm.at[src_v.at[j]], rows_v, sem).wait()
        pltpu.sync_copy(rows_v, acc.at[dst_v.at[j]], add=True)
        return carry

    lax.fori_loop(0, NCHUNK, body, 0)
    plsc.subcore_barrier()
    pltpu.sync_copy(acc.at[pl.ds(r0, RPT)], out_hbm.at[c, pl.ds(r0, RPT)])


def _make_agg(D):
    return pl.kernel(
        functools.partial(_agg_body, D),
        out_type=jax.ShapeDtypeStruct((NC, N, D), jnp.float32),
        mesh=_MESH,
        scratch_types=[
            pltpu.VMEM((NCHUNK, CHUNK), jnp.int32),
            pltpu.VMEM((NCHUNK, CHUNK), jnp.int32),
            pltpu.VMEM((CHUNK, D), jnp.float32),
            pltpu.VMEM((ZROWS, D), jnp.float32),
            pltpu.VMEM_SHARED((N, D), jnp.float32),
            pltpu.SemaphoreType.DMA,
        ],
    )


_agg16 = _make_agg(16)
_agg32 = _make_agg(32)


_BR = 1000


def _t1_body(deg_ref, x_ref, w_ref, dis_ref, g_ref):
    deg = deg_ref[0, :, 0:1] + deg_ref[1, :, 0:1] + 1.0
    dis = lax.rsqrt(deg)
    dis_ref[...] = dis
    g_ref[...] = dis * jnp.dot(x_ref[...], w_ref[...],
                               preferred_element_type=jnp.float32)


def _t1(degs, x, w1p):
    return pl.pallas_call(
        _t1_body,
        grid=(N // _BR,),
        in_specs=[
            pl.BlockSpec((NC, _BR, 16), lambda i: (0, i, 0)),
            pl.BlockSpec((_BR, 128), lambda i: (i, 0)),
            pl.BlockSpec((128, 16), lambda i: (0, 0)),
        ],
        out_specs=[
            pl.BlockSpec((_BR, 1), lambda i: (i, 0)),
            pl.BlockSpec((_BR, 16), lambda i: (i, 0)),
        ],
        out_shape=[
            jax.ShapeDtypeStruct((N, 1), jnp.float32),
            jax.ShapeDtypeStruct((N, 16), jnp.float32),
        ],
    )(degs, x, w1p)


def _t2_body(s_ref, g_ref, dis_ref, b_ref, out_ref):
    dis = dis_ref[...]
    u = dis * (s_ref[0] + s_ref[1] + g_ref[...])
    out_ref[...] = dis * _selu(u + b_ref[...])


def _t2(sagg, g, dis, bp):
    D = g.shape[1]
    return pl.pallas_call(
        _t2_body,
        grid=(N // _BR,),
        in_specs=[
            pl.BlockSpec((NC, _BR, D), lambda i: (0, i, 0)),
            pl.BlockSpec((_BR, D), lambda i: (i, 0)),
            pl.BlockSpec((_BR, 1), lambda i: (i, 0)),
            pl.BlockSpec((1, D), lambda i: (0, 0)),
        ],
        out_specs=pl.BlockSpec((_BR, D), lambda i: (i, 0)),
        out_shape=jax.ShapeDtypeStruct((N, D), jnp.float32),
    )(sagg, g, dis, bp)


def _t3_body(scale_out, s_ref, g_ref, dis_ref, w_ref, b_ref, out_ref):
    dis = dis_ref[...]
    u = dis * (s_ref[0] + s_ref[1] + g_ref[...])
    z = _selu(jnp.dot(u, w_ref[...], preferred_element_type=jnp.float32)
              + b_ref[...])
    out_ref[...] = dis * z if scale_out else z


def _t3(sagg, g, dis, wp, bp, scale_out=True):
    Din = g.shape[1]
    Dout = wp.shape[1]
    return pl.pallas_call(
        functools.partial(_t3_body, scale_out),
        grid=(N // _BR,),
        in_specs=[
            pl.BlockSpec((NC, _BR, Din), lambda i: (0, i, 0)),
            pl.BlockSpec((_BR, Din), lambda i: (i, 0)),
            pl.BlockSpec((_BR, 1), lambda i: (i, 0)),
            pl.BlockSpec((Din, Dout), lambda i: (0, 0)),
            pl.BlockSpec((1, Dout), lambda i: (0, 0)),
        ],
        out_specs=pl.BlockSpec((_BR, Dout), lambda i: (i, 0)),
        out_shape=jax.ShapeDtypeStruct((N, Dout), jnp.float32),
    )(sagg, g, dis, wp, bp)


def _pad(a, shape):
    return jnp.pad(a, [(0, t - c) for c, t in zip(a.shape, shape)])


def kernel(x, edge_index, W1, b1, W2, b2, W3, b3, W4, b4):
    ei = edge_index.astype(jnp.int32)
    src = ei[0].reshape(NW, NCHUNK, CHUNK)
    dst = ei[1].reshape(NW, NCHUNK, CHUNK)

    w1p = _pad(W1, (128, 16))
    b1p = _pad(b1, (16,)).reshape(1, 16)
    w2p = _pad(W2, (16, 32))
    b2p = _pad(b2, (32,)).reshape(1, 32)
    w3p = _pad(W3, (32, 32))
    b3p = _pad(b3, (32,)).reshape(1, 32)
    w4p = _pad(W4, (32, 36))
    b4p = b4.reshape(1, 36)

    degs = _deg_call(dst)
    dis, g1 = _t1(degs, x, w1p)
    s1 = _agg16(g1, src, dst)
    g2 = _t2(s1, g1, dis, b1p)
    s2 = _agg16(g2, src, dst)
    g3 = _t3(s2, g2, dis, w2p, b2p, scale_out=True)
    s3 = _agg32(g3, src, dst)
    g4 = _t3(s3, g3, dis, w3p, b3p, scale_out=True)
    s4 = _agg32(g4, src, dst)
    out = _t3(s4, g4, dis, w4p, b4p, scale_out=False)
    return out


# speedup vs baseline: 28.1833x; 28.1833x over previous
import functools

import jax
import jax.numpy as jnp
from jax import lax
from jax.experimental import pallas as pl
from jax.experimental.pallas import tpu as pltpu
from jax.experimental.pallas import tpu_sc as plsc

N = 10000
E = 320000
NC = 2
NS = 16
NW = NC * NS
EPW = E // NW
CHUNK = 125
NCHUNK = EPW // CHUNK
RPT = N // NS
ZROWS = 125

_MESH = plsc.VectorSubcoreMesh(core_axis_name="c", subcore_axis_name="s")


def _selu(x):
    alpha = 1.6732632423543772848170429916717
    scale = 1.0507009873554804934193349852946
    return scale * jnp.where(x > 0, x, alpha * (jnp.exp(x) - 1.0))


def _deg_body(dst_hbm, out_hbm, dst_v, ones_v, acc):
    c = lax.axis_index("c")
    s = lax.axis_index("s")
    wid = c * NS + s
    r0 = s * RPT

    def zfill(i, carry):
        ones_v[i, :] = jnp.zeros((16,), jnp.float32)
        return carry

    lax.fori_loop(0, ZROWS, zfill, 0)
    for k in range(RPT // ZROWS):
        pltpu.sync_copy(ones_v, acc.at[pl.ds(r0 + k * ZROWS, ZROWS)])

    def fill(i, carry):
        ones_v[i, :] = jnp.ones((16,), jnp.float32)
        return carry

    lax.fori_loop(0, CHUNK, fill, 0)
    pltpu.sync_copy(dst_hbm.at[wid], dst_v)
    plsc.subcore_barrier()

    def body(j, carry):
        pltpu.sync_copy(ones_v, acc.at[dst_v.at[j]], add=True)
        return carry

    lax.fori_loop(0, NCHUNK, body, 0)
    plsc.subcore_barrier()
    pltpu.sync_copy(acc.at[pl.ds(r0, RPT)], out_hbm.at[c, s])


_SC_PARAMS = pltpu.CompilerParams(use_tc_tiling_on_sc=False)

_deg_call = pl.kernel(
    _deg_body,
    out_type=jax.ShapeDtypeStruct((NC, NS, RPT, 16), jnp.float32),
    mesh=_MESH,
    compiler_params=_SC_PARAMS,
    scratch_types=[
        pltpu.VMEM((NCHUNK, CHUNK), jnp.int32),
        pltpu.VMEM((CHUNK, 16), jnp.float32),
        pltpu.VMEM_SHARED((N, 16), jnp.float32),
    ],
)


def _agg_body(D, g_hbm, src_hbm, dst_hbm, out_hbm, src_v, dst_v, rows_v, zbuf,
              acc, sem):
    c = lax.axis_index("c")
    s = lax.axis_index("s")
    wid = c * NS + s
    r0 = s * RPT

    def zfill(i, carry):
        for t in range(D // 16):
            zbuf[i, pl.ds(t * 16, 16)] = jnp.zeros((16,), jnp.float32)
        return carry

    lax.fori_loop(0, ZROWS, zfill, 0)
    for k in range(RPT // ZROWS):
        pltpu.sync_copy(zbuf, acc.at[pl.ds(r0 + k * ZROWS, ZROWS)])
    pltpu.sync_copy(src_hbm.at[wid], src_v)
    pltpu.sync_copy(dst_hbm.at[wid], dst_v)
    plsc.subcore_barrier()

    def body(j, carry):
        pltpu.async_copy(g_hbm.at[src_v.at[j]], rows_v, sem).wait()
        pltpu.sync_copy(rows_v, acc.at[dst_v.at[j]], add=True)
        return carry

    lax.fori_loop(0, NCHUNK, body, 0)
    plsc.subcore_barrier()
    pltpu.sync_copy(acc.at[pl.ds(r0, RPT)], out_hbm.at[c, s])


def _make_agg(D):
    return pl.kernel(
        functools.partial(_agg_body, D),
        out_type=jax.ShapeDtypeStruct((NC, NS, RPT, D), jnp.float32),
        mesh=_MESH,
        compiler_params=_SC_PARAMS,
        scratch_types=[
            pltpu.VMEM((NCHUNK, CHUNK), jnp.int32),
            pltpu.VMEM((NCHUNK, CHUNK), jnp.int32),
            pltpu.VMEM((CHUNK, D), jnp.float32),
            pltpu.VMEM((ZROWS, D), jnp.float32),
            pltpu.VMEM_SHARED((N, D), jnp.float32),
            pltpu.SemaphoreType.DMA,
        ],
    )


_agg16 = _make_agg(16)
_agg32 = _make_agg(32)


_BR = 1000


def _t1_body(deg_ref, x_ref, w_ref, dis_ref, g_ref):
    deg = deg_ref[0, :, 0:1] + deg_ref[1, :, 0:1] + 1.0
    dis = lax.rsqrt(deg)
    dis_ref[...] = dis
    g_ref[...] = dis * jnp.dot(x_ref[...], w_ref[...],
                               preferred_element_type=jnp.float32)


def _t1(degs, x, w1p):
    return pl.pallas_call(
        _t1_body,
        grid=(N // _BR,),
        in_specs=[
            pl.BlockSpec((NC, _BR, 16), lambda i: (0, i, 0)),
            pl.BlockSpec((_BR, 128), lambda i: (i, 0)),
            pl.BlockSpec((128, 16), lambda i: (0, 0)),
        ],
        out_specs=[
            pl.BlockSpec((_BR, 1), lambda i: (i, 0)),
            pl.BlockSpec((_BR, 16), lambda i: (i, 0)),
        ],
        out_shape=[
            jax.ShapeDtypeStruct((N, 1), jnp.float32),
            jax.ShapeDtypeStruct((N, 16), jnp.float32),
        ],
    )(degs, x, w1p)


def _t2_body(s_ref, g_ref, dis_ref, b_ref, out_ref):
    dis = dis_ref[...]
    u = dis * (s_ref[0] + s_ref[1] + g_ref[...])
    out_ref[...] = dis * _selu(u + b_ref[...])


def _t2(sagg, g, dis, bp):
    D = g.shape[1]
    return pl.pallas_call(
        _t2_body,
        grid=(N // _BR,),
        in_specs=[
            pl.BlockSpec((NC, _BR, D), lambda i: (0, i, 0)),
            pl.BlockSpec((_BR, D), lambda i: (i, 0)),
            pl.BlockSpec((_BR, 1), lambda i: (i, 0)),
            pl.BlockSpec((1, D), lambda i: (0, 0)),
        ],
        out_specs=pl.BlockSpec((_BR, D), lambda i: (i, 0)),
        out_shape=jax.ShapeDtypeStruct((N, D), jnp.float32),
    )(sagg, g, dis, bp)


def _t3_body(scale_out, s_ref, g_ref, dis_ref, w_ref, b_ref, out_ref):
    dis = dis_ref[...]
    u = dis * (s_ref[0] + s_ref[1] + g_ref[...])
    z = _selu(jnp.dot(u, w_ref[...], preferred_element_type=jnp.float32)
              + b_ref[...])
    out_ref[...] = dis * z if scale_out else z


def _t3(sagg, g, dis, wp, bp, scale_out=True):
    Din = g.shape[1]
    Dout = wp.shape[1]
    return pl.pallas_call(
        functools.partial(_t3_body, scale_out),
        grid=(N // _BR,),
        in_specs=[
            pl.BlockSpec((NC, _BR, Din), lambda i: (0, i, 0)),
            pl.BlockSpec((_BR, Din), lambda i: (i, 0)),
            pl.BlockSpec((_BR, 1), lambda i: (i, 0)),
            pl.BlockSpec((Din, Dout), lambda i: (0, 0)),
            pl.BlockSpec((1, Dout), lambda i: (0, 0)),
        ],
        out_specs=pl.BlockSpec((_BR, Dout), lambda i: (i, 0)),
        out_shape=jax.ShapeDtypeStruct((N, Dout), jnp.float32),
    )(sagg, g, dis, wp, bp)


def _pad(a, shape):
    return jnp.pad(a, [(0, t - c) for c, t in zip(a.shape, shape)])


def kernel(x, edge_index, W1, b1, W2, b2, W3, b3, W4, b4):
    ei = edge_index.astype(jnp.int32)
    src = ei[0].reshape(NW, NCHUNK, CHUNK)
    dst = ei[1].reshape(NW, NCHUNK, CHUNK)

    w1p = _pad(W1, (128, 16))
    b1p = _pad(b1, (16,)).reshape(1, 16)
    w2p = _pad(W2, (16, 32))
    b2p = _pad(b2, (32,)).reshape(1, 32)
    w3p = _pad(W3, (32, 32))
    b3p = _pad(b3, (32,)).reshape(1, 32)
    w4p = _pad(W4, (32, 36))
    b4p = b4.reshape(1, 36)

    degs = _deg_call(dst).reshape(NC, N, 16)
    dis, g1 = _t1(degs, x, w1p)
    s1 = _agg16(g1, src, dst).reshape(NC, N, 16)
    g2 = _t2(s1, g1, dis, b1p)
    s2 = _agg16(g2, src, dst).reshape(NC, N, 16)
    g3 = _t3(s2, g2, dis, w2p, b2p, scale_out=True)
    s3 = _agg32(g3, src, dst).reshape(NC, N, 32)
    g4 = _t3(s3, g3, dis, w3p, b3p, scale_out=True)
    s4 = _agg32(g4, src, dst).reshape(NC, N, 32)
    out = _t3(s4, g4, dis, w4p, b4p, scale_out=False)
    return out
